# Initial kernel scaffold; baseline (speedup 1.0000x reference)
import functools

import jax
import jax.numpy as jnp
from jax import lax
from jax.experimental import pallas as pl
from jax.experimental.pallas import tpu as pltpu
from jax.experimental.pallas import tpu_sc as plsc

H = 16
C = 16
G = 64

NC = 2
NS = 16
BLK = 80

f32 = jnp.float32


def _fold(W, att):
  D = W.shape[0]
  return jnp.einsum("dhc,hc->dh", W.reshape(D, H, C), att)


def _node_prep_body(x_ref, wip_ref, wts_ref, msi_ref, mdi_ref, mst_ref,
                    mdt_ref, hiplo_ref, hiphi_ref, htslo_ref, htshi_ref,
                    asi_ref, adi_ref, ast_ref, adt_ref):
  xb = x_ref[...]
  hip = jnp.dot(xb, wip_ref[...], preferred_element_type=f32)
  hiplo_ref[...] = hip[:, :128]
  hiphi_ref[...] = hip[:, 128:]
  hts = jnp.dot(xb, wts_ref[...], preferred_element_type=f32)
  htslo_ref[...] = hts[:, :128]
  htshi_ref[...] = hts[:, 128:]
  asi_ref[...] = jnp.dot(xb, msi_ref[...], preferred_element_type=f32)
  adi_ref[...] = jnp.dot(xb, mdi_ref[...], preferred_element_type=f32)
  ast_ref[...] = jnp.dot(xb, mst_ref[...], preferred_element_type=f32)
  adt_ref[...] = jnp.dot(xb, mdt_ref[...], preferred_element_type=f32)


def _node_prep(x, w_ip, w_ts, m_si, m_di, m_st, m_dt):
  N, D = x.shape
  R = 1000
  grid = N // R
  full = lambda shape: pl.BlockSpec(shape, lambda i: (0,) * len(shape))
  row = lambda w: pl.BlockSpec((R, w), lambda i: (i, 0))
  return pl.pallas_call(
      _node_prep_body,
      grid=(grid,),
      in_specs=[row(D), full((D, 256)), full((D, 256)), full((D, H)),
                full((D, H)), full((D, H)), full((D, H))],
      out_specs=[row(128), row(128), row(128), row(128),
                 row(H), row(H), row(H), row(H)],
      out_shape=[jax.ShapeDtypeStruct((N, 128), f32)] * 4
      + [jax.ShapeDtypeStruct((N, H), f32)] * 4,
  )(x, w_ip, w_ts, m_si, m_di, m_st, m_dt)


def _edge_prep_body(ea_ref, m_ref, out_ref):
  out_ref[...] = jnp.dot(ea_ref[...], m_ref[...], preferred_element_type=f32)


def _edge_prep(ea, m_edge):
  E, K = ea.shape
  R = 1280
  return pl.pallas_call(
      _edge_prep_body,
      grid=(E // R,),
      in_specs=[pl.BlockSpec((R, K), lambda i: (i, 0)),
                pl.BlockSpec((K, H), lambda i: (0, 0))],
      out_specs=pl.BlockSpec((R, H), lambda i: (i, 0)),
      out_shape=jax.ShapeDtypeStruct((E, H), f32),
  )(ea, m_edge)


@functools.lru_cache(maxsize=None)
def _make_sc_kernel(N, E):
  EPW = E // NS
  NBLK = EPW // BLK
  STRIPE = N // NS
  NZ = STRIPE // 125

  mesh = plsc.VectorSubcoreMesh(
      core_axis_name="c", subcore_axis_name="s", num_cores=NC,
      num_subcores=NS)

  def body(src_ip, dst_ip, ae_ip, asp_ip, adp_ip, hlo_ip, hhi_ip,
           src_ts, dst_ts, ae_ts, asp_ts, adp_ts, hlo_ts, hhi_ts,
           mlo_ip, mhi_ip, mlo_ts, mhi_ts, st_ip, st_ts,
           acc_sh, stats_sh, srcv, dstv, av, bv, ev, hv, sv, zb1, zb2,
           sem0, sem1, sem2, sem3):
    c = lax.axis_index("c")
    s = lax.axis_index("s")
    zero16 = jnp.zeros((16,), f32)
    one16 = jnp.ones((16,), f32)

    def z1(i, carry):
      zb1[i // 8, pl.ds((i % 8) * 16, 16)] = zero16
      return carry

    lax.fori_loop(0, 125 * 8, z1, 0)

    def z2(i, carry):
      zb2[i // 3, pl.ds((i % 3) * 16, 16)] = zero16
      return carry

    lax.fori_loop(0, 125 * 3, z2, 0)

    def prefill(j, carry):
      sv[j, pl.ds(32, 16)] = one16
      return carry

    lax.fori_loop(0, BLK, prefill, 0)

    lanevecs = [jnp.full((16,), c * 8 + hh, dtype=jnp.int32)
                for hh in range(8)]
    r0 = s * STRIPE

    rels = [
        (src_ip, dst_ip, ae_ip, asp_ip, adp_ip, hlo_ip, hhi_ip,
         mlo_ip, mhi_ip, st_ip),
        (src_ts, dst_ts, ae_ts, asp_ts, adp_ts, hlo_ts, hhi_ts,
         mlo_ts, mhi_ts, st_ts),
    ]
    for rel, (src_h, dst_h, ae_h, asp_h, adp_h, hlo_h, hhi_h,
              mlo_o, mhi_o, st_o) in enumerate(rels):
      owner = rel

      for k in range(NZ):
        pltpu.sync_copy(zb1, acc_sh.at[pl.ds(r0 + k * 125, 125)])

      @pl.when(c == owner)
      def _():
        for k in range(NZ):
          pltpu.sync_copy(zb2, stats_sh.at[pl.ds(r0 + k * 125, 125)])

      plsc.subcore_barrier()

      def block_body(b, carry):
        base = s * EPW + b * BLK
        pltpu.sync_copy(src_h.at[pl.ds(base, BLK)], srcv)
        pltpu.sync_copy(dst_h.at[pl.ds(base, BLK)], dstv)
        cp0 = pltpu.async_copy(asp_h.at[srcv], av, sem0)
        cp1 = pltpu.async_copy(adp_h.at[dstv], bv, sem1)
        cp2 = pltpu.async_copy(ae_h.at[pl.ds(base, BLK)], ev, sem2)

        @pl.when(c == 0)
        def _():
          pltpu.async_copy(hlo_h.at[srcv], hv, sem3).wait()

        @pl.when(c == 1)
        def _():
          pltpu.async_copy(hhi_h.at[srcv], hv, sem3).wait()

        cp0.wait()
        cp1.wait()
        cp2.wait()

        def edge_body(j, carry2):
          erow = ev[j]
          logit = av[j] + bv[j] + erow
          logit = jnp.maximum(logit, 0.2 * logit)
          p = jnp.exp(logit)
          sv[j, pl.ds(0, 16)] = p
          sv[j, pl.ds(16, 16)] = erow
          for hh in range(8):
            ph = jnp.take_along_axis(
                p, lanevecs[hh], axis=0,
                mode=lax.GatherScatterMode.PROMISE_IN_BOUNDS)
            off = hh * 16
            hv[j, pl.ds(off, 16)] = hv[j, pl.ds(off, 16)] * ph
          return carry2

        lax.fori_loop(0, BLK, edge_body, 0)

        pltpu.sync_copy(hv, acc_sh.at[dstv], add=True)

        @pl.when(c == owner)
        def _():
          pltpu.sync_copy(sv, stats_sh.at[dstv], add=True)

        return carry

      lax.fori_loop(0, NBLK, block_body, 0)
      plsc.subcore_barrier()

      @pl.when(c == 0)
      def _():
        pltpu.sync_copy(acc_sh.at[pl.ds(r0, STRIPE)],
                        mlo_o.at[pl.ds(r0, STRIPE)])

      @pl.when(c == 1)
      def _():
        pltpu.sync_copy(acc_sh.at[pl.ds(r0, STRIPE)],
                        mhi_o.at[pl.ds(r0, STRIPE)])

      @pl.when(c == owner)
      def _():
        pltpu.sync_copy(stats_sh.at[pl.ds(r0, STRIPE)],
                        st_o.at[pl.ds(r0, STRIPE)])

      plsc.subcore_barrier()

  return pl.kernel(
      body,
      out_type=(
          jax.ShapeDtypeStruct((N, 128), f32),
          jax.ShapeDtypeStruct((N, 128), f32),
          jax.ShapeDtypeStruct((N, 128), f32),
          jax.ShapeDtypeStruct((N, 128), f32),
          jax.ShapeDtypeStruct((N, 48), f32),
          jax.ShapeDtypeStruct((N, 48), f32),
      ),
      mesh=mesh,
      scratch_types=[
          pltpu.VMEM_SHARED((N, 128), f32),
          pltpu.VMEM_SHARED((N, 48), f32),
          pltpu.VMEM((BLK,), jnp.int32),
          pltpu.VMEM((BLK,), jnp.int32),
          pltpu.VMEM((BLK, 16), f32),
          pltpu.VMEM((BLK, 16), f32),
          pltpu.VMEM((BLK, 16), f32),
          pltpu.VMEM((BLK, 128), f32),
          pltpu.VMEM((BLK, 48), f32),
          pltpu.VMEM((125, 128), f32),
          pltpu.VMEM((125, 48), f32),
          pltpu.SemaphoreType.DMA,
          pltpu.SemaphoreType.DMA,
          pltpu.SemaphoreType.DMA,
          pltpu.SemaphoreType.DMA,
      ],
  )


def _finish_body(mlo_ip, mhi_ip, st_ip, hlo_ip, hhi_ip, asi, adi,
                 mlo_ts, mhi_ts, st_ts, hlo_ts, hhi_ts, ast, adt,
                 batch_ref, erep_ref, bgi_ref, bgt_ref,
                 wfa_ref, wfb_ref, bf_ref, w1_ref, b1_ref, w2_ref, b2_ref,
                 w3_ref, b3_ref, wev_ref, bev_ref, wen_ref, ben_ref,
                 oev_ref, oen_ref, acc_ref):
  i = pl.program_id(0)
  nsteps = pl.num_programs(0)
  erep = erep_ref[...]

  def gat_nodes(mlo, mhi, st, hlo, hhi, a_s, a_d, bg):
    stats = st[...]
    deg = jnp.maximum(stats[:, 32:33], 1.0)
    aeloop = stats[:, 16:32] / deg
    sl = a_s[...] + a_d[...] + aeloop
    p_loop = jnp.exp(jnp.maximum(sl, 0.2 * sl))
    denom = stats[:, 0:16] + p_loop + 1e-16
    msg = jnp.concatenate([mlo[...], mhi[...]], axis=1)
    hfull = jnp.concatenate([hlo[...], hhi[...]], axis=1)
    pexp = jnp.dot(p_loop, erep, preferred_element_type=f32)
    dexp = jnp.dot(denom, erep, preferred_element_type=f32)
    return (msg + pexp * hfull) / dexp + bg[...]

  nodes_ip = gat_nodes(mlo_ip, mhi_ip, st_ip, hlo_ip, hhi_ip, asi, adi,
                       bgi_ref)
  nodes_ts = gat_nodes(mlo_ts, mhi_ts, st_ts, hlo_ts, hhi_ts, ast, adt,
                       bgt_ref)
  f = jnp.dot(nodes_ip, wfa_ref[...], preferred_element_type=f32)
  f = f + jnp.dot(nodes_ts, wfb_ref[...], preferred_element_type=f32)
  f = jnp.maximum(f + bf_ref[...], 0.0)
  f = jnp.maximum(jnp.dot(f, w1_ref[...], preferred_element_type=f32)
                  + b1_ref[...], 0.0)
  f = jnp.maximum(jnp.dot(f, w2_ref[...], preferred_element_type=f32)
                  + b2_ref[...], 0.0)
  f = jnp.maximum(jnp.dot(f, w3_ref[...], preferred_element_type=f32)
                  + b3_ref[...], 0.0)

  bt = batch_ref[0, 0, :]
  R = f.shape[0]
  oht = (bt[None, :] == lax.broadcasted_iota(jnp.int32, (G, R), 0))
  f17 = jnp.concatenate([f, jnp.ones((R, 16), f32)], axis=1)
  contrib = jnp.dot(oht.astype(f32), f17, preferred_element_type=f32)

  @pl.when(i == 0)
  def _():
    acc_ref[...] = jnp.zeros_like(acc_ref)

  acc_ref[...] += contrib

  @pl.when(i == nsteps - 1)
  def _():
    acc = acc_ref[...]
    gf = acc[:, :16] / jnp.maximum(acc[:, 16:32], 1.0)
    oev_ref[...] = jnp.dot(gf, wev_ref[...],
                           preferred_element_type=f32) + bev_ref[...]
    oen_ref[...] = jnp.dot(gf, wen_ref[...],
                           preferred_element_type=f32) + ben_ref[...]


def _finish(mlo_ip, mhi_ip, st_ip, hlo_ip, hhi_ip, asi, adi,
            mlo_ts, mhi_ts, st_ts, hlo_ts, hhi_ts, ast, adt,
            batch, erep, bgi, bgt, wfa, wfb, bf, w1, b1, w2, b2, w3, b3,
            wev, bev, wen, ben):
  N = mlo_ip.shape[0]
  R = 1000
  grid = N // R
  batch3 = batch.reshape(grid, 1, R)
  full = lambda a: pl.BlockSpec(a.shape, lambda i: (0,) * a.ndim)
  row = lambda w: pl.BlockSpec((R, w), lambda i: (i, 0))
  node_in = [row(128), row(128), row(48), row(128), row(128), row(H), row(H)]
  return pl.pallas_call(
      _finish_body,
      grid=(grid,),
      in_specs=node_in + node_in + [
          pl.BlockSpec((1, 1, R), lambda i: (i, 0, 0)),
          full(erep), full(bgi), full(bgt), full(wfa), full(wfb), full(bf),
          full(w1), full(b1), full(w2), full(b2), full(w3), full(b3),
          full(wev), full(bev), full(wen), full(ben)],
      out_specs=[pl.BlockSpec((G, 3), lambda i: (0, 0)),
                 pl.BlockSpec((G, 3), lambda i: (0, 0))],
      out_shape=[jax.ShapeDtypeStruct((G, 3), f32),
                 jax.ShapeDtypeStruct((G, 3), f32)],
      scratch_shapes=[pltpu.VMEM((G, 32), f32)],
  )(mlo_ip, mhi_ip, st_ip, hlo_ip, hhi_ip, asi, adi,
    mlo_ts, mhi_ts, st_ts, hlo_ts, hhi_ts, ast, adt,
    batch3, erep, bgi, bgt, wfa, wfb, bf, w1, b1, w2, b2, w3, b3,
    wev, bev, wen, ben)


def kernel(x, edge_index_ipv6, edge_attr_ipv6, edge_index_tsch,
           edge_attr_tsch, batch, W_gat_ipv6, att_src_ipv6, att_dst_ipv6,
           W_edge_ipv6, att_edge_ipv6, b_gat_ipv6, W_gat_tsch, att_src_tsch,
           att_dst_tsch, W_edge_tsch, att_edge_tsch, b_gat_tsch, W_fuse,
           b_fuse, W_h1, b_h1, W_h2, b_h2, W_h3, b_h3, W_event, b_event,
           W_env, b_env):
  N = x.shape[0]
  E = edge_index_ipv6.shape[1]

  m_si = _fold(W_gat_ipv6, att_src_ipv6)
  m_di = _fold(W_gat_ipv6, att_dst_ipv6)
  m_ei = _fold(W_edge_ipv6, att_edge_ipv6)
  m_st = _fold(W_gat_tsch, att_src_tsch)
  m_dt = _fold(W_gat_tsch, att_dst_tsch)
  m_et = _fold(W_edge_tsch, att_edge_tsch)
  erep = jnp.kron(jnp.eye(H, dtype=f32), jnp.ones((1, C), f32))

  src_ip = edge_index_ipv6[0]
  dst_ip = edge_index_ipv6[1]
  src_ts = edge_index_tsch[0]
  dst_ts = edge_index_tsch[1]

  (hip_lo, hip_hi, hts_lo, hts_hi, asi, adi, ast, adt) = _node_prep(
      x, W_gat_ipv6, W_gat_tsch, m_si, m_di, m_st, m_dt)
  ae_ip = _edge_prep(edge_attr_ipv6, m_ei)
  ae_ts = _edge_prep(edge_attr_tsch, m_et)

  (mlo_ip, mhi_ip, mlo_ts, mhi_ts, st_ip, st_ts) = _make_sc_kernel(N, E)(
      src_ip, dst_ip, ae_ip, asi, adi, hip_lo, hip_hi,
      src_ts, dst_ts, ae_ts, ast, adt, hts_lo, hts_hi)

  out_event, out_env = _finish(
      mlo_ip, mhi_ip, st_ip, hip_lo, hip_hi, asi, adi,
      mlo_ts, mhi_ts, st_ts, hts_lo, hts_hi, ast, adt,
      batch, erep, b_gat_ipv6.reshape(1, -1), b_gat_tsch.reshape(1, -1),
      W_fuse[:256], W_fuse[256:], b_fuse.reshape(1, -1),
      W_h1, b_h1.reshape(1, -1), W_h2, b_h2.reshape(1, -1),
      W_h3, b_h3.reshape(1, -1), W_event, b_event.reshape(1, -1),
      W_env, b_env.reshape(1, -1))
  return (out_event, out_env)


# speedup vs baseline: 28.6552x; 28.6552x over previous
import functools

import jax
import jax.numpy as jnp
from jax import lax
from jax.experimental import pallas as pl
from jax.experimental.pallas import tpu as pltpu
from jax.experimental.pallas import tpu_sc as plsc

H = 16
C = 16
G = 64

NC = 2
NS = 16
BLK = 80

f32 = jnp.float32


def _fold(W, att):
  D = W.shape[0]
  return jnp.einsum("dhc,hc->dh", W.reshape(D, H, C), att)


def _node_prep_body(x_ref, wip_ref, wts_ref, msi_ref, mdi_ref, mst_ref,
                    mdt_ref, hiplo_ref, hiphi_ref, htslo_ref, htshi_ref,
                    asi_ref, adi_ref, ast_ref, adt_ref):
  xb = x_ref[...]
  hip = jnp.dot(xb, wip_ref[...], preferred_element_type=f32)
  hiplo_ref[...] = hip[:, :128]
  hiphi_ref[...] = hip[:, 128:]
  hts = jnp.dot(xb, wts_ref[...], preferred_element_type=f32)
  htslo_ref[...] = hts[:, :128]
  htshi_ref[...] = hts[:, 128:]
  asi_ref[...] = jnp.dot(xb, msi_ref[...], preferred_element_type=f32)
  adi_ref[...] = jnp.dot(xb, mdi_ref[...], preferred_element_type=f32)
  ast_ref[...] = jnp.dot(xb, mst_ref[...], preferred_element_type=f32)
  adt_ref[...] = jnp.dot(xb, mdt_ref[...], preferred_element_type=f32)


def _node_prep(x, w_ip, w_ts, m_si, m_di, m_st, m_dt):
  N, D = x.shape
  R = 1000
  grid = N // R
  full = lambda shape: pl.BlockSpec(shape, lambda i: (0,) * len(shape))
  row = lambda w: pl.BlockSpec((R, w), lambda i: (i, 0))
  return pl.pallas_call(
      _node_prep_body,
      grid=(grid,),
      in_specs=[row(D), full((D, 256)), full((D, 256)), full((D, H)),
                full((D, H)), full((D, H)), full((D, H))],
      out_specs=[row(128), row(128), row(128), row(128),
                 row(H), row(H), row(H), row(H)],
      out_shape=[jax.ShapeDtypeStruct((N, 128), f32)] * 4
      + [jax.ShapeDtypeStruct((N, H), f32)] * 4,
  )(x, w_ip, w_ts, m_si, m_di, m_st, m_dt)


def _edge_prep_body(ea_ref, m_ref, out_ref):
  out_ref[...] = jnp.dot(ea_ref[...], m_ref[...], preferred_element_type=f32)


def _edge_prep(ea, m_edge):
  E, K = ea.shape
  R = 1280
  return pl.pallas_call(
      _edge_prep_body,
      grid=(E // R,),
      in_specs=[pl.BlockSpec((R, K), lambda i: (i, 0)),
                pl.BlockSpec((K, H), lambda i: (0, 0))],
      out_specs=pl.BlockSpec((R, H), lambda i: (i, 0)),
      out_shape=jax.ShapeDtypeStruct((E, H), f32),
  )(ea, m_edge)


@functools.lru_cache(maxsize=None)
def _make_sc_kernel(N, E):
  EPW = E // NS
  NBLK = EPW // BLK
  STRIPE = -(-(N // NS) // 8) * 8
  NZ = -(-STRIPE // 16)
  NP = (NS - 1) * STRIPE + NZ * 16

  mesh = plsc.VectorSubcoreMesh(
      core_axis_name="c", subcore_axis_name="s", num_cores=NC,
      num_subcores=NS)

  def body(src_ip, dst_ip, ae_ip, asp_ip, adp_ip, hlo_ip, hhi_ip,
           src_ts, dst_ts, ae_ts, asp_ts, adp_ts, hlo_ts, hhi_ts,
           mlo_ip, mhi_ip, mlo_ts, mhi_ts, stp_ip, stp_ts, deg_ip, deg_ts,
           acc_sh, stats_sh, srcv, dstv, av, bv, ev, hv, sv, zb1, zb2,
           sem0, sem1, sem2, sem3):
    c = lax.axis_index("c")
    s = lax.axis_index("s")
    zero16 = jnp.zeros((16,), f32)
    one16 = jnp.ones((16,), f32)

    def z1(i, carry):
      zb1[i // 8, pl.ds((i % 8) * 16, 16)] = zero16
      return carry

    lax.fori_loop(0, 16 * 8, z1, 0)

    def z2(i, carry):
      zb2[i // 2, pl.ds((i % 2) * 16, 16)] = zero16
      return carry

    lax.fori_loop(0, 16 * 2, z2, 0)

    lanevecs = [jnp.full((16,), c * 8 + hh, dtype=jnp.int32)
                for hh in range(8)]
    r0 = s * STRIPE

    rels = [
        (src_ip, dst_ip, ae_ip, asp_ip, adp_ip, hlo_ip, hhi_ip,
         mlo_ip, mhi_ip, stp_ip, deg_ip),
        (src_ts, dst_ts, ae_ts, asp_ts, adp_ts, hlo_ts, hhi_ts,
         mlo_ts, mhi_ts, stp_ts, deg_ts),
    ]
    for rel, (src_h, dst_h, ae_h, asp_h, adp_h, hlo_h, hhi_h,
              mlo_o, mhi_o, stp_o, deg_o) in enumerate(rels):
      owner = rel

      for k in range(NZ):
        pltpu.sync_copy(zb1, acc_sh.at[pl.ds(r0 + k * 16, 16)])
        pltpu.sync_copy(zb2, stats_sh.at[pl.ds(r0 + k * 16, 16)])

      @pl.when(c == 1 - owner)
      def _():
        def prefill(j, carry):
          sv[j, pl.ds(0, 16)] = one16
          sv[j, pl.ds(16, 16)] = zero16
          return carry

        lax.fori_loop(0, BLK, prefill, 0)

      plsc.subcore_barrier()

      def block_body(b, carry):
        base = s * EPW + b * BLK
        pltpu.sync_copy(src_h.at[pl.ds(base, BLK)], srcv)
        pltpu.sync_copy(dst_h.at[pl.ds(base, BLK)], dstv)
        cp0 = pltpu.async_copy(asp_h.at[srcv], av, sem0)
        cp1 = pltpu.async_copy(adp_h.at[dstv], bv, sem1)
        cp2 = pltpu.async_copy(ae_h.at[pl.ds(base, BLK)], ev, sem2)

        @pl.when(c == 0)
        def _():
          pltpu.async_copy(hlo_h.at[srcv], hv, sem3).wait()

        @pl.when(c == 1)
        def _():
          pltpu.async_copy(hhi_h.at[srcv], hv, sem3).wait()

        cp0.wait()
        cp1.wait()
        cp2.wait()

        is_owner = c == owner

        def edge_body(j, carry2):
          erow = ev[j]
          logit = av[j] + bv[j] + erow
          logit = jnp.maximum(logit, 0.2 * logit)
          p = jnp.exp(logit)

          @pl.when(is_owner)
          def _():
            sv[j, pl.ds(0, 16)] = p
            sv[j, pl.ds(16, 16)] = erow

          for hh in range(8):
            ph = jnp.take_along_axis(
                p, lanevecs[hh], axis=0,
                mode=lax.GatherScatterMode.PROMISE_IN_BOUNDS)
            off = hh * 16
            hv[j, pl.ds(off, 16)] = hv[j, pl.ds(off, 16)] * ph
          return carry2

        lax.fori_loop(0, BLK, edge_body, 0)

        pltpu.sync_copy(hv, acc_sh.at[dstv], add=True)
        pltpu.sync_copy(sv, stats_sh.at[dstv], add=True)
        return carry

      lax.fori_loop(0, NBLK, block_body, 0)
      plsc.subcore_barrier()

      @pl.when(c == 0)
      def _():
        pltpu.sync_copy(acc_sh.at[pl.ds(r0, STRIPE)],
                        mlo_o.at[pl.ds(r0, STRIPE)])

      @pl.when(c == 1)
      def _():
        pltpu.sync_copy(acc_sh.at[pl.ds(r0, STRIPE)],
                        mhi_o.at[pl.ds(r0, STRIPE)])

      @pl.when(c == owner)
      def _():
        pltpu.sync_copy(stats_sh.at[pl.ds(r0, STRIPE)],
                        stp_o.at[pl.ds(r0, STRIPE)])

      @pl.when(c == 1 - owner)
      def _():
        pltpu.sync_copy(stats_sh.at[pl.ds(r0, STRIPE)],
                        deg_o.at[pl.ds(r0, STRIPE)])

      plsc.subcore_barrier()

  return pl.kernel(
      body,
      out_type=(
          jax.ShapeDtypeStruct((NP, 128), f32),
          jax.ShapeDtypeStruct((NP, 128), f32),
          jax.ShapeDtypeStruct((NP, 128), f32),
          jax.ShapeDtypeStruct((NP, 128), f32),
          jax.ShapeDtypeStruct((NP, 32), f32),
          jax.ShapeDtypeStruct((NP, 32), f32),
          jax.ShapeDtypeStruct((NP, 32), f32),
          jax.ShapeDtypeStruct((NP, 32), f32),
      ),
      mesh=mesh,
      scratch_types=[
          pltpu.VMEM_SHARED((NP, 128), f32),
          pltpu.VMEM_SHARED((NP, 32), f32),
          pltpu.VMEM((BLK,), jnp.int32),
          pltpu.VMEM((BLK,), jnp.int32),
          pltpu.VMEM((BLK, 16), f32),
          pltpu.VMEM((BLK, 16), f32),
          pltpu.VMEM((BLK, 16), f32),
          pltpu.VMEM((BLK, 128), f32),
          pltpu.VMEM((BLK, 32), f32),
          pltpu.VMEM((16, 128), f32),
          pltpu.VMEM((16, 32), f32),
          pltpu.SemaphoreType.DMA,
          pltpu.SemaphoreType.DMA,
          pltpu.SemaphoreType.DMA,
          pltpu.SemaphoreType.DMA,
      ],
      compiler_params=pltpu.CompilerParams(use_tc_tiling_on_sc=False),
  )


def _finish_body(mlo_ip, mhi_ip, stp_ip, deg_ip, hlo_ip, hhi_ip, asi, adi,
                 mlo_ts, mhi_ts, stp_ts, deg_ts, hlo_ts, hhi_ts, ast, adt,
                 batch_ref, erep_ref, bgi_ref, bgt_ref,
                 wfa_ref, wfb_ref, bf_ref, w1_ref, b1_ref, w2_ref, b2_ref,
                 w3_ref, b3_ref, wev_ref, bev_ref, wen_ref, ben_ref,
                 oev_ref, oen_ref, acc_ref):
  i = pl.program_id(0)
  nsteps = pl.num_programs(0)
  erep = erep_ref[...]

  def gat_nodes(mlo, mhi, stp, dg, hlo, hhi, a_s, a_d, bg):
    stats = stp[...]
    deg = jnp.maximum(dg[...][:, 0:1], 1.0)
    aeloop = stats[:, 16:32] / deg
    sl = a_s[...] + a_d[...] + aeloop
    p_loop = jnp.exp(jnp.maximum(sl, 0.2 * sl))
    denom = stats[:, 0:16] + p_loop + 1e-16
    msg = jnp.concatenate([mlo[...], mhi[...]], axis=1)
    hfull = jnp.concatenate([hlo[...], hhi[...]], axis=1)
    pexp = jnp.dot(p_loop, erep, preferred_element_type=f32)
    dexp = jnp.dot(denom, erep, preferred_element_type=f32)
    return (msg + pexp * hfull) / dexp + bg[...]

  nodes_ip = gat_nodes(mlo_ip, mhi_ip, stp_ip, deg_ip, hlo_ip, hhi_ip,
                       asi, adi, bgi_ref)
  nodes_ts = gat_nodes(mlo_ts, mhi_ts, stp_ts, deg_ts, hlo_ts, hhi_ts,
                       ast, adt, bgt_ref)
  f = jnp.dot(nodes_ip, wfa_ref[...], preferred_element_type=f32)
  f = f + jnp.dot(nodes_ts, wfb_ref[...], preferred_element_type=f32)
  f = jnp.maximum(f + bf_ref[...], 0.0)
  f = jnp.maximum(jnp.dot(f, w1_ref[...], preferred_element_type=f32)
                  + b1_ref[...], 0.0)
  f = jnp.maximum(jnp.dot(f, w2_ref[...], preferred_element_type=f32)
                  + b2_ref[...], 0.0)
  f = jnp.maximum(jnp.dot(f, w3_ref[...], preferred_element_type=f32)
                  + b3_ref[...], 0.0)

  bt = batch_ref[0, 0, :]
  R = f.shape[0]
  oht = (bt[None, :] == lax.broadcasted_iota(jnp.int32, (G, R), 0))
  f17 = jnp.concatenate([f, jnp.ones((R, 16), f32)], axis=1)
  contrib = jnp.dot(oht.astype(f32), f17, preferred_element_type=f32)

  @pl.when(i == 0)
  def _():
    acc_ref[...] = jnp.zeros_like(acc_ref)

  acc_ref[...] += contrib

  @pl.when(i == nsteps - 1)
  def _():
    acc = acc_ref[...]
    gf = acc[:, :16] / jnp.maximum(acc[:, 16:32], 1.0)
    oev_ref[...] = jnp.dot(gf, wev_ref[...],
                           preferred_element_type=f32) + bev_ref[...]
    oen_ref[...] = jnp.dot(gf, wen_ref[...],
                           preferred_element_type=f32) + ben_ref[...]


def _finish(mlo_ip, mhi_ip, stp_ip, deg_ip, hlo_ip, hhi_ip, asi, adi,
            mlo_ts, mhi_ts, stp_ts, deg_ts, hlo_ts, hhi_ts, ast, adt,
            batch, erep, bgi, bgt, wfa, wfb, bf, w1, b1, w2, b2, w3, b3,
            wev, bev, wen, ben):
  N = batch.shape[0]
  R = 1000
  grid = N // R
  batch3 = batch.reshape(grid, 1, R)
  full = lambda a: pl.BlockSpec(a.shape, lambda i: (0,) * a.ndim)
  row = lambda w: pl.BlockSpec((R, w), lambda i: (i, 0))
  node_in = [row(128), row(128), row(32), row(32), row(128), row(128),
             row(H), row(H)]
  return pl.pallas_call(
      _finish_body,
      grid=(grid,),
      in_specs=node_in + node_in + [
          pl.BlockSpec((1, 1, R), lambda i: (i, 0, 0)),
          full(erep), full(bgi), full(bgt), full(wfa), full(wfb), full(bf),
          full(w1), full(b1), full(w2), full(b2), full(w3), full(b3),
          full(wev), full(bev), full(wen), full(ben)],
      out_specs=[pl.BlockSpec((G, 3), lambda i: (0, 0)),
                 pl.BlockSpec((G, 3), lambda i: (0, 0))],
      out_shape=[jax.ShapeDtypeStruct((G, 3), f32),
                 jax.ShapeDtypeStruct((G, 3), f32)],
      scratch_shapes=[pltpu.VMEM((G, 32), f32)],
  )(mlo_ip, mhi_ip, stp_ip, deg_ip, hlo_ip, hhi_ip, asi, adi,
    mlo_ts, mhi_ts, stp_ts, deg_ts, hlo_ts, hhi_ts, ast, adt,
    batch3, erep, bgi, bgt, wfa, wfb, bf, w1, b1, w2, b2, w3, b3,
    wev, bev, wen, ben)


def kernel(x, edge_index_ipv6, edge_attr_ipv6, edge_index_tsch,
           edge_attr_tsch, batch, W_gat_ipv6, att_src_ipv6, att_dst_ipv6,
           W_edge_ipv6, att_edge_ipv6, b_gat_ipv6, W_gat_tsch, att_src_tsch,
           att_dst_tsch, W_edge_tsch, att_edge_tsch, b_gat_tsch, W_fuse,
           b_fuse, W_h1, b_h1, W_h2, b_h2, W_h3, b_h3, W_event, b_event,
           W_env, b_env):
  N = x.shape[0]
  E = edge_index_ipv6.shape[1]

  m_si = _fold(W_gat_ipv6, att_src_ipv6)
  m_di = _fold(W_gat_ipv6, att_dst_ipv6)
  m_ei = _fold(W_edge_ipv6, att_edge_ipv6)
  m_st = _fold(W_gat_tsch, att_src_tsch)
  m_dt = _fold(W_gat_tsch, att_dst_tsch)
  m_et = _fold(W_edge_tsch, att_edge_tsch)
  erep = jnp.kron(jnp.eye(H, dtype=f32), jnp.ones((1, C), f32))

  src_ip = edge_index_ipv6[0]
  dst_ip = edge_index_ipv6[1]
  src_ts = edge_index_tsch[0]
  dst_ts = edge_index_tsch[1]

  (hip_lo, hip_hi, hts_lo, hts_hi, asi, adi, ast, adt) = _node_prep(
      x, W_gat_ipv6, W_gat_tsch, m_si, m_di, m_st, m_dt)
  ae_ip = _edge_prep(edge_attr_ipv6, m_ei)
  ae_ts = _edge_prep(edge_attr_tsch, m_et)

  (mlo_ip, mhi_ip, mlo_ts, mhi_ts, stp_ip, stp_ts, deg_ip,
   deg_ts) = _make_sc_kernel(N, E)(
       src_ip, dst_ip, ae_ip, asi, adi, hip_lo, hip_hi,
       src_ts, dst_ts, ae_ts, ast, adt, hts_lo, hts_hi)

  out_event, out_env = _finish(
      mlo_ip, mhi_ip, stp_ip, deg_ip, hip_lo, hip_hi, asi, adi,
      mlo_ts, mhi_ts, stp_ts, deg_ts, hts_lo, hts_hi, ast, adt,
      batch, erep, b_gat_ipv6.reshape(1, -1), b_gat_tsch.reshape(1, -1),
      W_fuse[:256], W_fuse[256:], b_fuse.reshape(1, -1),
      W_h1, b_h1.reshape(1, -1), W_h2, b_h2.reshape(1, -1),
      W_h3, b_h3.reshape(1, -1), W_event, b_event.reshape(1, -1),
      W_env, b_env.reshape(1, -1))
  return (out_event, out_env)
